# baseline (speedup 1.0000x reference)
import functools

import jax
import jax.numpy as jnp
from jax import lax
from jax.experimental import pallas as pl
from jax.experimental.pallas import tpu as pltpu
from jax.experimental.pallas import tpu_sc as plsc

N_NODES = 10000
N_PAD = 10112
E_EDGES = 320000
K_CHUNK = 128
ROWS_PER_TILE = N_PAD // 16

D_FEAT = 128
H1 = 64
H2 = 32
N_CLASSES = 10

DEG_C0 = 120
DEG_C1 = 40
AGG_C0 = 160
AGG_C1 = 0
CH_TOTAL = 2560
CH_ALLOC = CH_TOTAL + max(DEG_C0, AGG_C0)
E_PAD = CH_ALLOC * K_CHUNK

_DEG_FLIGHT = 16
_P = 2
_NG = 3


@functools.cache
def _make_deg_kernel():
    @functools.partial(
        pl.kernel,
        mesh=plsc.VectorSubcoreMesh(core_axis_name="c", subcore_axis_name="s"),
        compiler_params=pltpu.CompilerParams(use_tc_tiling_on_sc=False),
        out_type=jax.ShapeDtypeStruct((2 * N_PAD, 8), jnp.float32),
        scratch_types=[
            pltpu.VMEM((DEG_C0, K_CHUNK), jnp.int32),
            pltpu.VMEM((K_CHUNK, 8), jnp.float32),
            pltpu.VMEM((ROWS_PER_TILE, 8), jnp.float32),
            pltpu.VMEM_SHARED((N_PAD, 8), jnp.float32),
            pltpu.SemaphoreType.DMA,
        ],
    )
    def deg_kernel(dst_hbm, ones_hbm, zeros_hbm, out_hbm,
                   dst_all, ones_v, wb_v, acc_sh, ssem):
        c = lax.axis_index("c")
        s = lax.axis_index("s")
        base = jnp.where(c == 0, s * DEG_C0, 16 * DEG_C0 + s * DEG_C1)
        nb = jnp.where(c == 0, DEG_C0, DEG_C1)
        pltpu.sync_copy(dst_hbm.at[pl.ds(base, DEG_C0)], dst_all)
        pltpu.sync_copy(ones_hbm, ones_v)
        pltpu.sync_copy(zeros_hbm, wb_v)
        pltpu.sync_copy(wb_v, acc_sh.at[pl.ds(s * ROWS_PER_TILE, ROWS_PER_TILE)])
        plsc.subcore_barrier()

        def body(ch, carry):
            pltpu.async_copy(ones_v, acc_sh.at[dst_all.at[ch]], ssem, add=True)

            @pl.when(ch >= _DEG_FLIGHT)
            def _():
                pltpu.make_async_copy(ones_v, acc_sh.at[dst_all.at[0]],
                                      ssem).wait()

            return carry

        lax.fori_loop(0, nb, body, 0)
        for _ in range(_DEG_FLIGHT):
            pltpu.make_async_copy(ones_v, acc_sh.at[dst_all.at[0]],
                                  ssem).wait()
        plsc.subcore_barrier()
        r0 = s * ROWS_PER_TILE
        pltpu.sync_copy(acc_sh.at[pl.ds(r0, ROWS_PER_TILE)], wb_v)
        pltpu.sync_copy(wb_v, out_hbm.at[pl.ds(c * N_PAD + r0, ROWS_PER_TILE)])

    return deg_kernel


@functools.cache
def _make_agg_kernel(h):
    @functools.partial(
        pl.kernel,
        mesh=plsc.VectorSubcoreMesh(core_axis_name="c", subcore_axis_name="s"),
        compiler_params=pltpu.CompilerParams(use_tc_tiling_on_sc=False),
        out_type=jax.ShapeDtypeStruct((2 * N_PAD, h), jnp.float32),
        scratch_types=[
            pltpu.VMEM((AGG_C0, K_CHUNK), jnp.int32),
            pltpu.VMEM((AGG_C0, K_CHUNK), jnp.int32),
            pltpu.VMEM((_NG * _P * K_CHUNK, h), jnp.float32),
            pltpu.VMEM_SHARED((N_PAD, h), jnp.float32),
            [pltpu.SemaphoreType.DMA] * _NG,
            [pltpu.SemaphoreType.DMA] * _NG,
        ],
    )
    def agg_kernel(y_hbm, src_hbm, dst_hbm, zeros_hbm, out_hbm,
                   src_all, dst_all, rows_v, acc_sh, gsems, ssems):
        c = lax.axis_index("c")
        s = lax.axis_index("s")
        base = jnp.where(c == 0, s * AGG_C0, 16 * AGG_C0 + s * AGG_C1)
        nchunks = jnp.where(c == 0, AGG_C0, AGG_C1)
        nb = nchunks // _P
        with jax.named_scope("agg_init"):
            pltpu.sync_copy(src_hbm.at[pl.ds(base, AGG_C0)], src_all)
            pltpu.sync_copy(dst_hbm.at[pl.ds(base, AGG_C0)], dst_all)
            pltpu.sync_copy(zeros_hbm, rows_v.at[pl.ds(0, ROWS_PER_TILE)])
            pltpu.sync_copy(rows_v.at[pl.ds(0, ROWS_PER_TILE)],
                            acc_sh.at[pl.ds(s * ROWS_PER_TILE, ROWS_PER_TILE)])
            plsc.subcore_barrier()

        def fire_gathers(b, g):
            for p in range(_P):
                ch = b * _P + p
                buf = (g * _P + p) * K_CHUNK
                pltpu.async_copy(y_hbm.at[src_all.at[ch]],
                                 rows_v.at[pl.ds(buf, K_CHUNK)], gsems[g])

        def drain_gathers(g):
            for p in range(_P):
                buf = (g * _P + p) * K_CHUNK
                pltpu.make_async_copy(y_hbm.at[src_all.at[0]],
                                      rows_v.at[pl.ds(buf, K_CHUNK)],
                                      gsems[g]).wait()

        def fire_scatters(b, g):
            for p in range(_P):
                ch = b * _P + p
                buf = (g * _P + p) * K_CHUNK
                pltpu.async_copy(rows_v.at[pl.ds(buf, K_CHUNK)],
                                 acc_sh.at[dst_all.at[ch]], ssems[g], add=True)

        def drain_scatters(g):
            for p in range(_P):
                buf = (g * _P + p) * K_CHUNK
                pltpu.make_async_copy(rows_v.at[pl.ds(buf, K_CHUNK)],
                                      acc_sh.at[dst_all.at[0]], ssems[g]).wait()

        @pl.when(nb >= 1)
        def _():
            fire_gathers(0, 0)

        @pl.when(nb >= 2)
        def _():
            fire_gathers(1, 1)

        def body(b, carry):
            for g in range(_NG):
                @pl.when(lax.rem(b, _NG) == g)
                def _(g=g):
                    drain_gathers(g)
                    fire_scatters(b, g)

                    @pl.when(b >= 1)
                    def _():
                        drain_scatters((g + 2) % _NG)

                    @pl.when(b + 2 < nb)
                    def _():
                        fire_gathers(b + 2, (g + 2) % _NG)

            return carry

        with jax.named_scope("agg_loop"):
            lax.fori_loop(0, nb, body, 0)
            for g in range(_NG):
                @pl.when((nb >= 1) & (lax.rem(nb - 1, _NG) == g))
                def _(g=g):
                    drain_scatters(g)
            plsc.subcore_barrier()
        with jax.named_scope("agg_wb"):
            r0 = s * ROWS_PER_TILE
            pltpu.sync_copy(acc_sh.at[pl.ds(r0, ROWS_PER_TILE)],
                            rows_v.at[pl.ds(0, ROWS_PER_TILE)])
            pltpu.sync_copy(rows_v.at[pl.ds(0, ROWS_PER_TILE)],
                            out_hbm.at[pl.ds(c * N_PAD + r0, ROWS_PER_TILE)])

    return agg_kernel


_BN = 1000


def _tc1_body(x_ref, w_ref, d0_ref, d1_ref, y_ref, dinv_ref):
    deg = d0_ref[:, :1] + d1_ref[:, :1] + 1.0
    dinv = lax.rsqrt(deg)
    h = jnp.dot(x_ref[...], w_ref[...], preferred_element_type=jnp.float32)
    y_ref[...] = h * dinv
    dinv_ref[...] = dinv


def _tc1(x, w1, d0, d1):
    return pl.pallas_call(
        _tc1_body,
        grid=(N_NODES // _BN,),
        in_specs=[
            pl.BlockSpec((_BN, D_FEAT), lambda i: (i, 0)),
            pl.BlockSpec((D_FEAT, H1), lambda i: (0, 0)),
            pl.BlockSpec((_BN, 8), lambda i: (i, 0)),
            pl.BlockSpec((_BN, 8), lambda i: (i, 0)),
        ],
        out_specs=[
            pl.BlockSpec((_BN, H1), lambda i: (i, 0)),
            pl.BlockSpec((_BN, 1), lambda i: (i, 0)),
        ],
        out_shape=[
            jax.ShapeDtypeStruct((N_NODES, H1), jnp.float32),
            jax.ShapeDtypeStruct((N_NODES, 1), jnp.float32),
        ],
    )(x, w1, d0, d1)


def _tc2_body(a0_ref, a1_ref, y1_ref, dinv_ref, b1_ref, w2_ref, y2_ref):
    agg = a0_ref[...] + a1_ref[...] + y1_ref[...]
    dinv = dinv_ref[...]
    hid = jnp.maximum(agg * dinv + b1_ref[...], 0.0)
    y2_ref[...] = jnp.dot(hid, w2_ref[...], preferred_element_type=jnp.float32) * dinv


def _tc2(a0, a1, y1, dinv, b1, w2):
    return pl.pallas_call(
        _tc2_body,
        grid=(N_NODES // _BN,),
        in_specs=[
            pl.BlockSpec((_BN, H1), lambda i: (i, 0)),
            pl.BlockSpec((_BN, H1), lambda i: (i, 0)),
            pl.BlockSpec((_BN, H1), lambda i: (i, 0)),
            pl.BlockSpec((_BN, 1), lambda i: (i, 0)),
            pl.BlockSpec((1, H1), lambda i: (0, 0)),
            pl.BlockSpec((H1, H2), lambda i: (0, 0)),
        ],
        out_specs=pl.BlockSpec((_BN, H2), lambda i: (i, 0)),
        out_shape=jax.ShapeDtypeStruct((N_NODES, H2), jnp.float32),
    )(a0, a1, y1, dinv, b1, w2)


def _tc3_body(a0_ref, a1_ref, y2_ref, dinv_ref, b2_ref, wc_ref, bc_ref, out_ref):
    agg = a0_ref[...] + a1_ref[...] + y2_ref[...]
    hid = jnp.maximum(agg * dinv_ref[...] + b2_ref[...], 0.0)
    logits = jnp.dot(hid, wc_ref[...], preferred_element_type=jnp.float32) + bc_ref[...]
    m = jnp.max(logits, axis=1, keepdims=True)
    lse = jnp.log(jnp.sum(jnp.exp(logits - m), axis=1, keepdims=True)) + m
    out_ref[...] = logits - lse


def _tc3(a0, a1, y2, dinv, b2, wc, bc):
    return pl.pallas_call(
        _tc3_body,
        grid=(N_NODES // _BN,),
        in_specs=[
            pl.BlockSpec((_BN, H2), lambda i: (i, 0)),
            pl.BlockSpec((_BN, H2), lambda i: (i, 0)),
            pl.BlockSpec((_BN, H2), lambda i: (i, 0)),
            pl.BlockSpec((_BN, 1), lambda i: (i, 0)),
            pl.BlockSpec((1, H2), lambda i: (0, 0)),
            pl.BlockSpec((H2, N_CLASSES), lambda i: (0, 0)),
            pl.BlockSpec((1, N_CLASSES), lambda i: (0, 0)),
        ],
        out_specs=pl.BlockSpec((_BN, N_CLASSES), lambda i: (i, 0)),
        out_shape=jax.ShapeDtypeStruct((N_NODES, N_CLASSES), jnp.float32),
    )(a0, a1, y2, dinv, b2, wc, bc)


def kernel(x, edge_index, W1, b1, W2, b2, Wc, bc):
    src = edge_index[0].astype(jnp.int32)
    dst = edge_index[1].astype(jnp.int32)
    pad = jnp.full((E_PAD - E_EDGES,), N_NODES, jnp.int32)
    src_p = jnp.concatenate([src, pad]).reshape(CH_ALLOC, K_CHUNK)
    dst_p = jnp.concatenate([dst, pad]).reshape(CH_ALLOC, K_CHUNK)

    ones8 = jnp.ones((K_CHUNK, 8), jnp.float32)
    zeros8 = jnp.zeros((ROWS_PER_TILE, 8), jnp.float32)
    zeros1 = jnp.zeros((ROWS_PER_TILE, H1), jnp.float32)
    zeros2 = jnp.zeros((ROWS_PER_TILE, H2), jnp.float32)

    degp = _make_deg_kernel()(dst_p, ones8, zeros8)
    d0 = degp[:N_NODES]
    d1 = degp[N_PAD:N_PAD + N_NODES]

    y1, dinv = _tc1(x, W1, d0, d1)
    y1p = jnp.concatenate(
        [y1, jnp.zeros((N_PAD - N_NODES, H1), jnp.float32)], axis=0)
    agg1 = _make_agg_kernel(H1)(y1p, src_p, dst_p, zeros1)
    y2 = _tc2(agg1[:N_NODES], agg1[N_PAD:N_PAD + N_NODES], y1, dinv,
              b1.reshape(1, H1), W2)
    y2p = jnp.concatenate(
        [y2, jnp.zeros((N_PAD - N_NODES, H2), jnp.float32)], axis=0)
    agg2 = _make_agg_kernel(H2)(y2p, src_p, dst_p, zeros2)
    return _tc3(agg2[:N_NODES], agg2[N_PAD:N_PAD + N_NODES], y2, dinv,
                b2.reshape(1, H2), Wc, bc.reshape(1, N_CLASSES))


# speedup vs baseline: 1.0358x; 1.0358x over previous
import functools

import jax
import jax.numpy as jnp
from jax import lax
from jax.experimental import pallas as pl
from jax.experimental.pallas import tpu as pltpu
from jax.experimental.pallas import tpu_sc as plsc

N_NODES = 10000
N_PAD = 10112
E_EDGES = 320000
K_CHUNK = 256
ROWS_PER_TILE = N_PAD // 16

D_FEAT = 128
H1 = 64
H2 = 32
N_CLASSES = 10

DEG_C0 = 40
DEG_C1 = 40
AGG_C0 = 40
AGG_C1 = 40
CH_ALLOC = 16 * (AGG_C0 + AGG_C1)
E_PAD = CH_ALLOC * K_CHUNK

_DEG_FLIGHT = 16
_P = 1
_NG = 3


@functools.cache
def _make_deg_kernel():
    @functools.partial(
        pl.kernel,
        mesh=plsc.VectorSubcoreMesh(core_axis_name="c", subcore_axis_name="s"),
        compiler_params=pltpu.CompilerParams(use_tc_tiling_on_sc=False),
        out_type=jax.ShapeDtypeStruct((2 * N_PAD, 8), jnp.float32),
        scratch_types=[
            pltpu.VMEM((DEG_C0, K_CHUNK), jnp.int32),
            pltpu.VMEM((K_CHUNK, 8), jnp.float32),
            pltpu.VMEM((ROWS_PER_TILE, 8), jnp.float32),
            pltpu.VMEM_SHARED((N_PAD, 8), jnp.float32),
            pltpu.SemaphoreType.DMA,
        ],
    )
    def deg_kernel(dst_hbm, ones_hbm, zeros_hbm, out_hbm,
                   dst_all, ones_v, wb_v, acc_sh, ssem):
        c = lax.axis_index("c")
        s = lax.axis_index("s")
        base = jnp.where(c == 0, s * DEG_C0, 16 * DEG_C0 + s * DEG_C1)
        nb = jnp.where(c == 0, DEG_C0, DEG_C1)
        pltpu.sync_copy(dst_hbm.at[pl.ds(base, DEG_C0)], dst_all)
        pltpu.sync_copy(ones_hbm, ones_v)
        pltpu.sync_copy(zeros_hbm, wb_v)
        pltpu.sync_copy(wb_v, acc_sh.at[pl.ds(s * ROWS_PER_TILE, ROWS_PER_TILE)])
        plsc.subcore_barrier()

        def body(ch, carry):
            pltpu.async_copy(ones_v, acc_sh.at[dst_all.at[ch]], ssem, add=True)

            @pl.when(ch >= _DEG_FLIGHT)
            def _():
                pltpu.make_async_copy(ones_v, acc_sh.at[dst_all.at[0]],
                                      ssem).wait()

            return carry

        lax.fori_loop(0, nb, body, 0)
        for _ in range(_DEG_FLIGHT):
            pltpu.make_async_copy(ones_v, acc_sh.at[dst_all.at[0]],
                                  ssem).wait()
        plsc.subcore_barrier()
        r0 = s * ROWS_PER_TILE
        pltpu.sync_copy(acc_sh.at[pl.ds(r0, ROWS_PER_TILE)], wb_v)
        pltpu.sync_copy(wb_v, out_hbm.at[pl.ds(c * N_PAD + r0, ROWS_PER_TILE)])

    return deg_kernel


@functools.cache
def _make_agg_kernel(h):
    @functools.partial(
        pl.kernel,
        mesh=plsc.VectorSubcoreMesh(core_axis_name="c", subcore_axis_name="s"),
        compiler_params=pltpu.CompilerParams(use_tc_tiling_on_sc=False),
        out_type=jax.ShapeDtypeStruct((2 * N_PAD, h), jnp.float32),
        scratch_types=[
            pltpu.VMEM((AGG_C0, K_CHUNK), jnp.int32),
            pltpu.VMEM((AGG_C0, K_CHUNK), jnp.int32),
            pltpu.VMEM((_NG * _P * K_CHUNK, h), jnp.float32),
            pltpu.VMEM_SHARED((N_PAD, h), jnp.float32),
            [pltpu.SemaphoreType.DMA] * _NG,
            [pltpu.SemaphoreType.DMA] * _NG,
        ],
    )
    def agg_kernel(y_hbm, src_hbm, dst_hbm, zeros_hbm, out_hbm,
                   src_all, dst_all, rows_v, acc_sh, gsems, ssems):
        c = lax.axis_index("c")
        s = lax.axis_index("s")
        base = jnp.where(c == 0, s * AGG_C0, 16 * AGG_C0 + s * AGG_C1)
        nchunks = jnp.where(c == 0, AGG_C0, AGG_C1)
        nb = nchunks // _P
        with jax.named_scope("agg_init"):
            pltpu.sync_copy(src_hbm.at[pl.ds(base, AGG_C0)], src_all)
            pltpu.sync_copy(dst_hbm.at[pl.ds(base, AGG_C0)], dst_all)
            pltpu.sync_copy(zeros_hbm, rows_v.at[pl.ds(0, ROWS_PER_TILE)])
            pltpu.sync_copy(rows_v.at[pl.ds(0, ROWS_PER_TILE)],
                            acc_sh.at[pl.ds(s * ROWS_PER_TILE, ROWS_PER_TILE)])
            plsc.subcore_barrier()

        def fire_gathers(b, g):
            for p in range(_P):
                ch = b * _P + p
                buf = (g * _P + p) * K_CHUNK
                pltpu.async_copy(y_hbm.at[src_all.at[ch]],
                                 rows_v.at[pl.ds(buf, K_CHUNK)], gsems[g])

        def drain_gathers(g):
            for p in range(_P):
                buf = (g * _P + p) * K_CHUNK
                pltpu.make_async_copy(y_hbm.at[src_all.at[0]],
                                      rows_v.at[pl.ds(buf, K_CHUNK)],
                                      gsems[g]).wait()

        def fire_scatters(b, g):
            for p in range(_P):
                ch = b * _P + p
                buf = (g * _P + p) * K_CHUNK
                pltpu.async_copy(rows_v.at[pl.ds(buf, K_CHUNK)],
                                 acc_sh.at[dst_all.at[ch]], ssems[g], add=True)

        def drain_scatters(g):
            for p in range(_P):
                buf = (g * _P + p) * K_CHUNK
                pltpu.make_async_copy(rows_v.at[pl.ds(buf, K_CHUNK)],
                                      acc_sh.at[dst_all.at[0]], ssems[g]).wait()

        @pl.when(nb >= 1)
        def _():
            fire_gathers(0, 0)

        @pl.when(nb >= 2)
        def _():
            fire_gathers(1, 1)

        def body(b, carry):
            for g in range(_NG):
                @pl.when(lax.rem(b, _NG) == g)
                def _(g=g):
                    drain_gathers(g)
                    fire_scatters(b, g)

                    @pl.when(b >= 1)
                    def _():
                        drain_scatters((g + 2) % _NG)

                    @pl.when(b + 2 < nb)
                    def _():
                        fire_gathers(b + 2, (g + 2) % _NG)

            return carry

        with jax.named_scope("agg_loop"):
            lax.fori_loop(0, nb, body, 0)
            for g in range(_NG):
                @pl.when((nb >= 1) & (lax.rem(nb - 1, _NG) == g))
                def _(g=g):
                    drain_scatters(g)
            plsc.subcore_barrier()
        with jax.named_scope("agg_wb"):
            r0 = s * ROWS_PER_TILE
            pltpu.sync_copy(acc_sh.at[pl.ds(r0, ROWS_PER_TILE)],
                            rows_v.at[pl.ds(0, ROWS_PER_TILE)])
            pltpu.sync_copy(rows_v.at[pl.ds(0, ROWS_PER_TILE)],
                            out_hbm.at[pl.ds(c * N_PAD + r0, ROWS_PER_TILE)])

    return agg_kernel


_BN = 1000


def _tc1_body(x_ref, w_ref, d0_ref, d1_ref, y_ref, dinv_ref):
    deg = d0_ref[:, :1] + d1_ref[:, :1] + 1.0
    dinv = lax.rsqrt(deg)
    h = jnp.dot(x_ref[...], w_ref[...], preferred_element_type=jnp.float32)
    y_ref[...] = h * dinv
    dinv_ref[...] = dinv


def _tc1(x, w1, d0, d1):
    return pl.pallas_call(
        _tc1_body,
        grid=(N_NODES // _BN,),
        in_specs=[
            pl.BlockSpec((_BN, D_FEAT), lambda i: (i, 0)),
            pl.BlockSpec((D_FEAT, H1), lambda i: (0, 0)),
            pl.BlockSpec((_BN, 8), lambda i: (i, 0)),
            pl.BlockSpec((_BN, 8), lambda i: (i, 0)),
        ],
        out_specs=[
            pl.BlockSpec((_BN, H1), lambda i: (i, 0)),
            pl.BlockSpec((_BN, 1), lambda i: (i, 0)),
        ],
        out_shape=[
            jax.ShapeDtypeStruct((N_NODES, H1), jnp.float32),
            jax.ShapeDtypeStruct((N_NODES, 1), jnp.float32),
        ],
    )(x, w1, d0, d1)


def _tc2_body(a0_ref, a1_ref, y1_ref, dinv_ref, b1_ref, w2_ref, y2_ref):
    agg = a0_ref[...] + a1_ref[...] + y1_ref[...]
    dinv = dinv_ref[...]
    hid = jnp.maximum(agg * dinv + b1_ref[...], 0.0)
    y2_ref[...] = jnp.dot(hid, w2_ref[...], preferred_element_type=jnp.float32) * dinv


def _tc2(a0, a1, y1, dinv, b1, w2):
    return pl.pallas_call(
        _tc2_body,
        grid=(N_NODES // _BN,),
        in_specs=[
            pl.BlockSpec((_BN, H1), lambda i: (i, 0)),
            pl.BlockSpec((_BN, H1), lambda i: (i, 0)),
            pl.BlockSpec((_BN, H1), lambda i: (i, 0)),
            pl.BlockSpec((_BN, 1), lambda i: (i, 0)),
            pl.BlockSpec((1, H1), lambda i: (0, 0)),
            pl.BlockSpec((H1, H2), lambda i: (0, 0)),
        ],
        out_specs=pl.BlockSpec((_BN, H2), lambda i: (i, 0)),
        out_shape=jax.ShapeDtypeStruct((N_NODES, H2), jnp.float32),
    )(a0, a1, y1, dinv, b1, w2)


def _tc3_body(a0_ref, a1_ref, y2_ref, dinv_ref, b2_ref, wc_ref, bc_ref, out_ref):
    agg = a0_ref[...] + a1_ref[...] + y2_ref[...]
    hid = jnp.maximum(agg * dinv_ref[...] + b2_ref[...], 0.0)
    logits = jnp.dot(hid, wc_ref[...], preferred_element_type=jnp.float32) + bc_ref[...]
    m = jnp.max(logits, axis=1, keepdims=True)
    lse = jnp.log(jnp.sum(jnp.exp(logits - m), axis=1, keepdims=True)) + m
    out_ref[...] = logits - lse


def _tc3(a0, a1, y2, dinv, b2, wc, bc):
    return pl.pallas_call(
        _tc3_body,
        grid=(N_NODES // _BN,),
        in_specs=[
            pl.BlockSpec((_BN, H2), lambda i: (i, 0)),
            pl.BlockSpec((_BN, H2), lambda i: (i, 0)),
            pl.BlockSpec((_BN, H2), lambda i: (i, 0)),
            pl.BlockSpec((_BN, 1), lambda i: (i, 0)),
            pl.BlockSpec((1, H2), lambda i: (0, 0)),
            pl.BlockSpec((H2, N_CLASSES), lambda i: (0, 0)),
            pl.BlockSpec((1, N_CLASSES), lambda i: (0, 0)),
        ],
        out_specs=pl.BlockSpec((_BN, N_CLASSES), lambda i: (i, 0)),
        out_shape=jax.ShapeDtypeStruct((N_NODES, N_CLASSES), jnp.float32),
    )(a0, a1, y2, dinv, b2, wc, bc)


def kernel(x, edge_index, W1, b1, W2, b2, Wc, bc):
    src = edge_index[0].astype(jnp.int32)
    dst = edge_index[1].astype(jnp.int32)
    pad = jnp.full((E_PAD - E_EDGES,), N_NODES, jnp.int32)
    src_p = jnp.concatenate([src, pad]).reshape(CH_ALLOC, K_CHUNK)
    dst_p = jnp.concatenate([dst, pad]).reshape(CH_ALLOC, K_CHUNK)

    ones8 = jnp.ones((K_CHUNK, 8), jnp.float32)
    zeros8 = jnp.zeros((ROWS_PER_TILE, 8), jnp.float32)
    zeros1 = jnp.zeros((ROWS_PER_TILE, H1), jnp.float32)
    zeros2 = jnp.zeros((ROWS_PER_TILE, H2), jnp.float32)

    degp = _make_deg_kernel()(dst_p, ones8, zeros8)
    d0 = degp[:N_NODES]
    d1 = degp[N_PAD:N_PAD + N_NODES]

    y1, dinv = _tc1(x, W1, d0, d1)
    y1p = jnp.concatenate(
        [y1, jnp.zeros((N_PAD - N_NODES, H1), jnp.float32)], axis=0)
    agg1 = _make_agg_kernel(H1)(y1p, src_p, dst_p, zeros1)
    y2 = _tc2(agg1[:N_NODES], agg1[N_PAD:N_PAD + N_NODES], y1, dinv,
              b1.reshape(1, H1), W2)
    y2p = jnp.concatenate(
        [y2, jnp.zeros((N_PAD - N_NODES, H2), jnp.float32)], axis=0)
    agg2 = _make_agg_kernel(H2)(y2p, src_p, dst_p, zeros2)
    return _tc3(agg2[:N_NODES], agg2[N_PAD:N_PAD + N_NODES], y2, dinv,
                b2.reshape(1, H2), Wc, bc.reshape(1, N_CLASSES))


# speedup vs baseline: 1.5273x; 1.4745x over previous
import functools

import jax
import jax.numpy as jnp
from jax import lax
from jax.experimental import pallas as pl
from jax.experimental.pallas import tpu as pltpu
from jax.experimental.pallas import tpu_sc as plsc

N_NODES = 10000
N_PAD = 10112
E_EDGES = 320000
K_CHUNK = 256
ROWS_PER_TILE = N_PAD // 16

D_FEAT = 128
H1 = 64
H2 = 32
N_CLASSES = 10

DEG_C0 = 40
DEG_C1 = 40
AGG_C0 = 40
AGG_C1 = 40
CH_ALLOC = 16 * (AGG_C0 + AGG_C1)
E_PAD = CH_ALLOC * K_CHUNK

_DEG_FLIGHT = 16
_P = 1
_NG = 3


@functools.cache
def _make_deg_kernel():
    @functools.partial(
        pl.kernel,
        mesh=plsc.VectorSubcoreMesh(core_axis_name="c", subcore_axis_name="s"),
        compiler_params=pltpu.CompilerParams(use_tc_tiling_on_sc=False),
        out_type=jax.ShapeDtypeStruct((2 * N_PAD, 8), jnp.float32),
        scratch_types=[
            pltpu.VMEM((DEG_C0, K_CHUNK), jnp.int32),
            pltpu.VMEM((K_CHUNK, 8), jnp.float32),
            pltpu.VMEM((ROWS_PER_TILE, 8), jnp.float32),
            pltpu.VMEM_SHARED((N_PAD, 8), jnp.float32),
            pltpu.SemaphoreType.DMA,
        ],
    )
    def deg_kernel(dst_hbm, ones_hbm, zeros_hbm, out_hbm,
                   dst_all, ones_v, wb_v, acc_sh, ssem):
        c = lax.axis_index("c")
        s = lax.axis_index("s")
        base = jnp.where(c == 0, s * DEG_C0, 16 * DEG_C0 + s * DEG_C1)
        nb = jnp.where(c == 0, DEG_C0, DEG_C1)
        pltpu.sync_copy(dst_hbm.at[pl.ds(base, DEG_C0)], dst_all)
        pltpu.sync_copy(ones_hbm, ones_v)
        pltpu.sync_copy(zeros_hbm, wb_v)
        pltpu.sync_copy(wb_v, acc_sh.at[pl.ds(s * ROWS_PER_TILE, ROWS_PER_TILE)])
        plsc.subcore_barrier()

        def body(ch, carry):
            pltpu.async_copy(ones_v, acc_sh.at[dst_all.at[ch]], ssem, add=True)

            @pl.when(ch >= _DEG_FLIGHT)
            def _():
                pltpu.make_async_copy(ones_v, acc_sh.at[dst_all.at[0]],
                                      ssem).wait()

            return carry

        lax.fori_loop(0, nb, body, 0)
        for _ in range(_DEG_FLIGHT):
            pltpu.make_async_copy(ones_v, acc_sh.at[dst_all.at[0]],
                                  ssem).wait()
        plsc.subcore_barrier()
        r0 = s * ROWS_PER_TILE
        pltpu.sync_copy(acc_sh.at[pl.ds(r0, ROWS_PER_TILE)], wb_v)
        pltpu.sync_copy(wb_v, out_hbm.at[pl.ds(c * N_PAD + r0, ROWS_PER_TILE)])

    return deg_kernel


@functools.cache
def _make_agg_kernel(h, dtype=jnp.bfloat16):
    @functools.partial(
        pl.kernel,
        mesh=plsc.VectorSubcoreMesh(core_axis_name="c", subcore_axis_name="s"),
        compiler_params=pltpu.CompilerParams(use_tc_tiling_on_sc=False),
        out_type=jax.ShapeDtypeStruct((2 * N_PAD, h), dtype),
        scratch_types=[
            pltpu.VMEM((AGG_C0, K_CHUNK), jnp.int32),
            pltpu.VMEM((AGG_C0, K_CHUNK), jnp.int32),
            pltpu.VMEM((_NG * _P * K_CHUNK, h), dtype),
            pltpu.VMEM_SHARED((N_PAD, h), dtype),
            [pltpu.SemaphoreType.DMA] * _NG,
            [pltpu.SemaphoreType.DMA] * _NG,
        ],
    )
    def agg_kernel(y_hbm, src_hbm, dst_hbm, zeros_hbm, out_hbm,
                   src_all, dst_all, rows_v, acc_sh, gsems, ssems):
        c = lax.axis_index("c")
        s = lax.axis_index("s")
        base = jnp.where(c == 0, s * AGG_C0, 16 * AGG_C0 + s * AGG_C1)
        nchunks = jnp.where(c == 0, AGG_C0, AGG_C1)
        nb = nchunks // _P
        with jax.named_scope("agg_init"):
            pltpu.sync_copy(src_hbm.at[pl.ds(base, AGG_C0)], src_all)
            pltpu.sync_copy(dst_hbm.at[pl.ds(base, AGG_C0)], dst_all)
            pltpu.sync_copy(zeros_hbm, rows_v.at[pl.ds(0, ROWS_PER_TILE)])
            pltpu.sync_copy(rows_v.at[pl.ds(0, ROWS_PER_TILE)],
                            acc_sh.at[pl.ds(s * ROWS_PER_TILE, ROWS_PER_TILE)])
            plsc.subcore_barrier()

        def fire_gathers(b, g):
            for p in range(_P):
                ch = b * _P + p
                buf = (g * _P + p) * K_CHUNK
                pltpu.async_copy(y_hbm.at[src_all.at[ch]],
                                 rows_v.at[pl.ds(buf, K_CHUNK)], gsems[g])

        def drain_gathers(g):
            for p in range(_P):
                buf = (g * _P + p) * K_CHUNK
                pltpu.make_async_copy(y_hbm.at[src_all.at[0]],
                                      rows_v.at[pl.ds(buf, K_CHUNK)],
                                      gsems[g]).wait()

        def fire_scatters(b, g):
            for p in range(_P):
                ch = b * _P + p
                buf = (g * _P + p) * K_CHUNK
                pltpu.async_copy(rows_v.at[pl.ds(buf, K_CHUNK)],
                                 acc_sh.at[dst_all.at[ch]], ssems[g], add=True)

        def drain_scatters(g):
            for p in range(_P):
                buf = (g * _P + p) * K_CHUNK
                pltpu.make_async_copy(rows_v.at[pl.ds(buf, K_CHUNK)],
                                      acc_sh.at[dst_all.at[0]], ssems[g]).wait()

        @pl.when(nb >= 1)
        def _():
            fire_gathers(0, 0)

        @pl.when(nb >= 2)
        def _():
            fire_gathers(1, 1)

        def body(b, carry):
            for g in range(_NG):
                @pl.when(lax.rem(b, _NG) == g)
                def _(g=g):
                    drain_gathers(g)
                    fire_scatters(b, g)

                    @pl.when(b >= 1)
                    def _():
                        drain_scatters((g + 2) % _NG)

                    @pl.when(b + 2 < nb)
                    def _():
                        fire_gathers(b + 2, (g + 2) % _NG)

            return carry

        with jax.named_scope("agg_loop"):
            lax.fori_loop(0, nb, body, 0)
            for g in range(_NG):
                @pl.when((nb >= 1) & (lax.rem(nb - 1, _NG) == g))
                def _(g=g):
                    drain_scatters(g)
            plsc.subcore_barrier()
        with jax.named_scope("agg_wb"):
            r0 = s * ROWS_PER_TILE
            pltpu.sync_copy(acc_sh.at[pl.ds(r0, ROWS_PER_TILE)],
                            rows_v.at[pl.ds(0, ROWS_PER_TILE)])
            pltpu.sync_copy(rows_v.at[pl.ds(0, ROWS_PER_TILE)],
                            out_hbm.at[pl.ds(c * N_PAD + r0, ROWS_PER_TILE)])

    return agg_kernel


_BN = 1000


def _tc1_body(x_ref, w_ref, d0_ref, d1_ref, y_ref, dinv_ref):
    deg = d0_ref[:, :1] + d1_ref[:, :1] + 1.0
    dinv = lax.rsqrt(deg)
    h = jnp.dot(x_ref[...], w_ref[...], preferred_element_type=jnp.float32)
    y_ref[...] = h * dinv
    dinv_ref[...] = dinv


def _tc1(x, w1, d0, d1):
    return pl.pallas_call(
        _tc1_body,
        grid=(N_NODES // _BN,),
        in_specs=[
            pl.BlockSpec((_BN, D_FEAT), lambda i: (i, 0)),
            pl.BlockSpec((D_FEAT, H1), lambda i: (0, 0)),
            pl.BlockSpec((_BN, 8), lambda i: (i, 0)),
            pl.BlockSpec((_BN, 8), lambda i: (i, 0)),
        ],
        out_specs=[
            pl.BlockSpec((_BN, H1), lambda i: (i, 0)),
            pl.BlockSpec((_BN, 1), lambda i: (i, 0)),
        ],
        out_shape=[
            jax.ShapeDtypeStruct((N_NODES, H1), jnp.float32),
            jax.ShapeDtypeStruct((N_NODES, 1), jnp.float32),
        ],
    )(x, w1, d0, d1)


def _tc2_body(a0_ref, a1_ref, y1_ref, dinv_ref, b1_ref, w2_ref, y2_ref):
    agg = a0_ref[...] + a1_ref[...] + y1_ref[...]
    dinv = dinv_ref[...]
    hid = jnp.maximum(agg * dinv + b1_ref[...], 0.0)
    y2_ref[...] = jnp.dot(hid, w2_ref[...], preferred_element_type=jnp.float32) * dinv


def _tc2(a0, a1, y1, dinv, b1, w2):
    return pl.pallas_call(
        _tc2_body,
        grid=(N_NODES // _BN,),
        in_specs=[
            pl.BlockSpec((_BN, H1), lambda i: (i, 0)),
            pl.BlockSpec((_BN, H1), lambda i: (i, 0)),
            pl.BlockSpec((_BN, H1), lambda i: (i, 0)),
            pl.BlockSpec((_BN, 1), lambda i: (i, 0)),
            pl.BlockSpec((1, H1), lambda i: (0, 0)),
            pl.BlockSpec((H1, H2), lambda i: (0, 0)),
        ],
        out_specs=pl.BlockSpec((_BN, H2), lambda i: (i, 0)),
        out_shape=jax.ShapeDtypeStruct((N_NODES, H2), jnp.float32),
    )(a0, a1, y1, dinv, b1, w2)


def _tc3_body(a0_ref, a1_ref, y2_ref, dinv_ref, b2_ref, wc_ref, bc_ref, out_ref):
    agg = a0_ref[...] + a1_ref[...] + y2_ref[...]
    hid = jnp.maximum(agg * dinv_ref[...] + b2_ref[...], 0.0)
    logits = jnp.dot(hid, wc_ref[...], preferred_element_type=jnp.float32) + bc_ref[...]
    m = jnp.max(logits, axis=1, keepdims=True)
    lse = jnp.log(jnp.sum(jnp.exp(logits - m), axis=1, keepdims=True)) + m
    out_ref[...] = logits - lse


def _tc3(a0, a1, y2, dinv, b2, wc, bc):
    return pl.pallas_call(
        _tc3_body,
        grid=(N_NODES // _BN,),
        in_specs=[
            pl.BlockSpec((_BN, H2), lambda i: (i, 0)),
            pl.BlockSpec((_BN, H2), lambda i: (i, 0)),
            pl.BlockSpec((_BN, H2), lambda i: (i, 0)),
            pl.BlockSpec((_BN, 1), lambda i: (i, 0)),
            pl.BlockSpec((1, H2), lambda i: (0, 0)),
            pl.BlockSpec((H2, N_CLASSES), lambda i: (0, 0)),
            pl.BlockSpec((1, N_CLASSES), lambda i: (0, 0)),
        ],
        out_specs=pl.BlockSpec((_BN, N_CLASSES), lambda i: (i, 0)),
        out_shape=jax.ShapeDtypeStruct((N_NODES, N_CLASSES), jnp.float32),
    )(a0, a1, y2, dinv, b2, wc, bc)


def kernel(x, edge_index, W1, b1, W2, b2, Wc, bc):
    src = edge_index[0].astype(jnp.int32)
    dst = edge_index[1].astype(jnp.int32)
    pad = jnp.full((E_PAD - E_EDGES,), N_NODES, jnp.int32)
    src_p = jnp.concatenate([src, pad]).reshape(CH_ALLOC, K_CHUNK)
    dst_p = jnp.concatenate([dst, pad]).reshape(CH_ALLOC, K_CHUNK)

    ones8 = jnp.ones((K_CHUNK, 8), jnp.float32)
    zeros8 = jnp.zeros((ROWS_PER_TILE, 8), jnp.float32)
    zeros1 = jnp.zeros((ROWS_PER_TILE, H1), jnp.bfloat16)
    zeros2 = jnp.zeros((ROWS_PER_TILE, H2), jnp.bfloat16)

    degp = _make_deg_kernel()(dst_p, ones8, zeros8)
    d0 = degp[:N_NODES]
    d1 = degp[N_PAD:N_PAD + N_NODES]

    y1, dinv = _tc1(x, W1, d0, d1)
    y1p = jnp.concatenate(
        [y1, jnp.zeros((N_NODES, H1), jnp.float32)[:N_PAD - N_NODES]],
        axis=0).astype(jnp.bfloat16)
    agg1 = _make_agg_kernel(H1)(y1p, src_p, dst_p, zeros1)
    y2 = _tc2(agg1[:N_NODES].astype(jnp.float32),
              agg1[N_PAD:N_PAD + N_NODES].astype(jnp.float32), y1, dinv,
              b1.reshape(1, H1), W2)
    y2p = jnp.concatenate(
        [y2, jnp.zeros((N_NODES, H2), jnp.float32)[:N_PAD - N_NODES]],
        axis=0).astype(jnp.bfloat16)
    agg2 = _make_agg_kernel(H2)(y2p, src_p, dst_p, zeros2)
    return _tc3(agg2[:N_NODES].astype(jnp.float32),
                agg2[N_PAD:N_PAD + N_NODES].astype(jnp.float32), y2, dinv,
                b2.reshape(1, H2), Wc, bc.reshape(1, N_CLASSES))


# speedup vs baseline: 1.5516x; 1.0159x over previous
import functools

import jax
import jax.numpy as jnp
from jax import lax
from jax.experimental import pallas as pl
from jax.experimental.pallas import tpu as pltpu
from jax.experimental.pallas import tpu_sc as plsc

N_NODES = 10000
N_PAD = 10112
E_EDGES = 320000
K_CHUNK = 256
ROWS_PER_TILE = N_PAD // 16

D_FEAT = 128
H1 = 64
H2 = 32
N_CLASSES = 10

DEG_C0 = 56
DEG_C1 = 24
AGG_C0 = 56
AGG_C1 = 24
CH_TOTAL = 16 * (AGG_C0 + AGG_C1)
CH_ALLOC = 16 * AGG_C0 + 15 * AGG_C1 + AGG_C0
E_PAD = CH_ALLOC * K_CHUNK

_DEG_FLIGHT = 16
_P = 1
_NG = 3


@functools.cache
def _make_deg_kernel():
    @functools.partial(
        pl.kernel,
        mesh=plsc.VectorSubcoreMesh(core_axis_name="c", subcore_axis_name="s"),
        compiler_params=pltpu.CompilerParams(use_tc_tiling_on_sc=False),
        out_type=jax.ShapeDtypeStruct((2 * N_PAD, 8), jnp.float32),
        scratch_types=[
            pltpu.VMEM((DEG_C0, K_CHUNK), jnp.int32),
            pltpu.VMEM((K_CHUNK, 8), jnp.float32),
            pltpu.VMEM((ROWS_PER_TILE, 8), jnp.float32),
            pltpu.VMEM_SHARED((N_PAD, 8), jnp.float32),
            pltpu.SemaphoreType.DMA,
        ],
    )
    def deg_kernel(dst_hbm, ones_hbm, zeros_hbm, out_hbm,
                   dst_all, ones_v, wb_v, acc_sh, ssem):
        c = lax.axis_index("c")
        s = lax.axis_index("s")
        base = jnp.where(c == 0, s * DEG_C0, 16 * DEG_C0 + s * DEG_C1)
        nb = jnp.where(c == 0, DEG_C0, DEG_C1)
        pltpu.sync_copy(dst_hbm.at[pl.ds(base, DEG_C0)], dst_all)
        pltpu.sync_copy(ones_hbm, ones_v)
        pltpu.sync_copy(zeros_hbm, wb_v)
        pltpu.sync_copy(wb_v, acc_sh.at[pl.ds(s * ROWS_PER_TILE, ROWS_PER_TILE)])
        plsc.subcore_barrier()

        def body(ch, carry):
            pltpu.async_copy(ones_v, acc_sh.at[dst_all.at[ch]], ssem, add=True)

            @pl.when(ch >= _DEG_FLIGHT)
            def _():
                pltpu.make_async_copy(ones_v, acc_sh.at[dst_all.at[0]],
                                      ssem).wait()

            return carry

        lax.fori_loop(0, nb, body, 0)
        for _ in range(_DEG_FLIGHT):
            pltpu.make_async_copy(ones_v, acc_sh.at[dst_all.at[0]],
                                  ssem).wait()
        plsc.subcore_barrier()
        r0 = s * ROWS_PER_TILE
        pltpu.sync_copy(acc_sh.at[pl.ds(r0, ROWS_PER_TILE)], wb_v)
        pltpu.sync_copy(wb_v, out_hbm.at[pl.ds(c * N_PAD + r0, ROWS_PER_TILE)])

    return deg_kernel


@functools.cache
def _make_agg_kernel(h, dtype=jnp.bfloat16):
    @functools.partial(
        pl.kernel,
        mesh=plsc.VectorSubcoreMesh(core_axis_name="c", subcore_axis_name="s"),
        compiler_params=pltpu.CompilerParams(use_tc_tiling_on_sc=False),
        out_type=jax.ShapeDtypeStruct((2 * N_PAD, h), dtype),
        scratch_types=[
            pltpu.VMEM((AGG_C0, K_CHUNK), jnp.int32),
            pltpu.VMEM((AGG_C0, K_CHUNK), jnp.int32),
            pltpu.VMEM((_NG * _P * K_CHUNK, h), dtype),
            pltpu.VMEM_SHARED((N_PAD, h), dtype),
            [pltpu.SemaphoreType.DMA] * _NG,
            [pltpu.SemaphoreType.DMA] * _NG,
        ],
    )
    def agg_kernel(y_hbm, src_hbm, dst_hbm, zeros_hbm, out_hbm,
                   src_all, dst_all, rows_v, acc_sh, gsems, ssems):
        c = lax.axis_index("c")
        s = lax.axis_index("s")
        base = jnp.where(c == 0, s * AGG_C0, 16 * AGG_C0 + s * AGG_C1)
        nchunks = jnp.where(c == 0, AGG_C0, AGG_C1)
        nb = nchunks // _P
        with jax.named_scope("agg_init"):
            pltpu.sync_copy(src_hbm.at[pl.ds(base, AGG_C0)], src_all)
            pltpu.sync_copy(dst_hbm.at[pl.ds(base, AGG_C0)], dst_all)
            pltpu.sync_copy(zeros_hbm, rows_v.at[pl.ds(0, ROWS_PER_TILE)])
            pltpu.sync_copy(rows_v.at[pl.ds(0, ROWS_PER_TILE)],
                            acc_sh.at[pl.ds(s * ROWS_PER_TILE, ROWS_PER_TILE)])
            plsc.subcore_barrier()

        def fire_gathers(b, g):
            for p in range(_P):
                ch = b * _P + p
                buf = (g * _P + p) * K_CHUNK
                pltpu.async_copy(y_hbm.at[src_all.at[ch]],
                                 rows_v.at[pl.ds(buf, K_CHUNK)], gsems[g])

        def drain_gathers(g):
            for p in range(_P):
                buf = (g * _P + p) * K_CHUNK
                pltpu.make_async_copy(y_hbm.at[src_all.at[0]],
                                      rows_v.at[pl.ds(buf, K_CHUNK)],
                                      gsems[g]).wait()

        def fire_scatters(b, g):
            for p in range(_P):
                ch = b * _P + p
                buf = (g * _P + p) * K_CHUNK
                pltpu.async_copy(rows_v.at[pl.ds(buf, K_CHUNK)],
                                 acc_sh.at[dst_all.at[ch]], ssems[g], add=True)

        def drain_scatters(g):
            for p in range(_P):
                buf = (g * _P + p) * K_CHUNK
                pltpu.make_async_copy(rows_v.at[pl.ds(buf, K_CHUNK)],
                                      acc_sh.at[dst_all.at[0]], ssems[g]).wait()

        @pl.when(nb >= 1)
        def _():
            fire_gathers(0, 0)

        @pl.when(nb >= 2)
        def _():
            fire_gathers(1, 1)

        def body(b, carry):
            for g in range(_NG):
                @pl.when(lax.rem(b, _NG) == g)
                def _(g=g):
                    drain_gathers(g)
                    fire_scatters(b, g)

                    @pl.when(b >= 1)
                    def _():
                        drain_scatters((g + 2) % _NG)

                    @pl.when(b + 2 < nb)
                    def _():
                        fire_gathers(b + 2, (g + 2) % _NG)

            return carry

        with jax.named_scope("agg_loop"):
            lax.fori_loop(0, nb, body, 0)
            for g in range(_NG):
                @pl.when((nb >= 1) & (lax.rem(nb - 1, _NG) == g))
                def _(g=g):
                    drain_scatters(g)
            plsc.subcore_barrier()
        with jax.named_scope("agg_wb"):
            r0 = s * ROWS_PER_TILE
            pltpu.sync_copy(acc_sh.at[pl.ds(r0, ROWS_PER_TILE)],
                            rows_v.at[pl.ds(0, ROWS_PER_TILE)])
            pltpu.sync_copy(rows_v.at[pl.ds(0, ROWS_PER_TILE)],
                            out_hbm.at[pl.ds(c * N_PAD + r0, ROWS_PER_TILE)])

    return agg_kernel


_BN = 1000


def _tc1_body(x_ref, w_ref, d0_ref, d1_ref, y_ref, dinv_ref):
    deg = d0_ref[:, :1] + d1_ref[:, :1] + 1.0
    dinv = lax.rsqrt(deg)
    h = jnp.dot(x_ref[...], w_ref[...], preferred_element_type=jnp.float32)
    y_ref[...] = h * dinv
    dinv_ref[...] = dinv


def _tc1(x, w1, d0, d1):
    return pl.pallas_call(
        _tc1_body,
        grid=(N_NODES // _BN,),
        in_specs=[
            pl.BlockSpec((_BN, D_FEAT), lambda i: (i, 0)),
            pl.BlockSpec((D_FEAT, H1), lambda i: (0, 0)),
            pl.BlockSpec((_BN, 8), lambda i: (i, 0)),
            pl.BlockSpec((_BN, 8), lambda i: (i, 0)),
        ],
        out_specs=[
            pl.BlockSpec((_BN, H1), lambda i: (i, 0)),
            pl.BlockSpec((_BN, 1), lambda i: (i, 0)),
        ],
        out_shape=[
            jax.ShapeDtypeStruct((N_NODES, H1), jnp.float32),
            jax.ShapeDtypeStruct((N_NODES, 1), jnp.float32),
        ],
    )(x, w1, d0, d1)


def _tc2_body(a0_ref, a1_ref, y1_ref, dinv_ref, b1_ref, w2_ref, y2_ref):
    agg = a0_ref[...] + a1_ref[...] + y1_ref[...]
    dinv = dinv_ref[...]
    hid = jnp.maximum(agg * dinv + b1_ref[...], 0.0)
    y2_ref[...] = jnp.dot(hid, w2_ref[...], preferred_element_type=jnp.float32) * dinv


def _tc2(a0, a1, y1, dinv, b1, w2):
    return pl.pallas_call(
        _tc2_body,
        grid=(N_NODES // _BN,),
        in_specs=[
            pl.BlockSpec((_BN, H1), lambda i: (i, 0)),
            pl.BlockSpec((_BN, H1), lambda i: (i, 0)),
            pl.BlockSpec((_BN, H1), lambda i: (i, 0)),
            pl.BlockSpec((_BN, 1), lambda i: (i, 0)),
            pl.BlockSpec((1, H1), lambda i: (0, 0)),
            pl.BlockSpec((H1, H2), lambda i: (0, 0)),
        ],
        out_specs=pl.BlockSpec((_BN, H2), lambda i: (i, 0)),
        out_shape=jax.ShapeDtypeStruct((N_NODES, H2), jnp.float32),
    )(a0, a1, y1, dinv, b1, w2)


def _tc3_body(a0_ref, a1_ref, y2_ref, dinv_ref, b2_ref, wc_ref, bc_ref, out_ref):
    agg = a0_ref[...] + a1_ref[...] + y2_ref[...]
    hid = jnp.maximum(agg * dinv_ref[...] + b2_ref[...], 0.0)
    logits = jnp.dot(hid, wc_ref[...], preferred_element_type=jnp.float32) + bc_ref[...]
    m = jnp.max(logits, axis=1, keepdims=True)
    lse = jnp.log(jnp.sum(jnp.exp(logits - m), axis=1, keepdims=True)) + m
    out_ref[...] = logits - lse


def _tc3(a0, a1, y2, dinv, b2, wc, bc):
    return pl.pallas_call(
        _tc3_body,
        grid=(N_NODES // _BN,),
        in_specs=[
            pl.BlockSpec((_BN, H2), lambda i: (i, 0)),
            pl.BlockSpec((_BN, H2), lambda i: (i, 0)),
            pl.BlockSpec((_BN, H2), lambda i: (i, 0)),
            pl.BlockSpec((_BN, 1), lambda i: (i, 0)),
            pl.BlockSpec((1, H2), lambda i: (0, 0)),
            pl.BlockSpec((H2, N_CLASSES), lambda i: (0, 0)),
            pl.BlockSpec((1, N_CLASSES), lambda i: (0, 0)),
        ],
        out_specs=pl.BlockSpec((_BN, N_CLASSES), lambda i: (i, 0)),
        out_shape=jax.ShapeDtypeStruct((N_NODES, N_CLASSES), jnp.float32),
    )(a0, a1, y2, dinv, b2, wc, bc)


def kernel(x, edge_index, W1, b1, W2, b2, Wc, bc):
    src = edge_index[0].astype(jnp.int32)
    dst = edge_index[1].astype(jnp.int32)
    pad = jnp.full((E_PAD - E_EDGES,), N_NODES, jnp.int32)
    src_p = jnp.concatenate([src, pad]).reshape(CH_ALLOC, K_CHUNK)
    dst_p = jnp.concatenate([dst, pad]).reshape(CH_ALLOC, K_CHUNK)

    ones8 = jnp.ones((K_CHUNK, 8), jnp.float32)
    zeros8 = jnp.zeros((ROWS_PER_TILE, 8), jnp.float32)
    zeros1 = jnp.zeros((ROWS_PER_TILE, H1), jnp.bfloat16)
    zeros2 = jnp.zeros((ROWS_PER_TILE, H2), jnp.bfloat16)

    degp = _make_deg_kernel()(dst_p, ones8, zeros8)
    d0 = degp[:N_NODES]
    d1 = degp[N_PAD:N_PAD + N_NODES]

    y1, dinv = _tc1(x, W1, d0, d1)
    y1p = jnp.concatenate(
        [y1, jnp.zeros((N_NODES, H1), jnp.float32)[:N_PAD - N_NODES]],
        axis=0).astype(jnp.bfloat16)
    agg1 = _make_agg_kernel(H1)(y1p, src_p, dst_p, zeros1)
    y2 = _tc2(agg1[:N_NODES].astype(jnp.float32),
              agg1[N_PAD:N_PAD + N_NODES].astype(jnp.float32), y1, dinv,
              b1.reshape(1, H1), W2)
    y2p = jnp.concatenate(
        [y2, jnp.zeros((N_NODES, H2), jnp.float32)[:N_PAD - N_NODES]],
        axis=0).astype(jnp.bfloat16)
    agg2 = _make_agg_kernel(H2)(y2p, src_p, dst_p, zeros2)
    return _tc3(agg2[:N_NODES].astype(jnp.float32),
                agg2[N_PAD:N_PAD + N_NODES].astype(jnp.float32), y2, dinv,
                b2.reshape(1, H2), Wc, bc.reshape(1, N_CLASSES))
